# Initial kernel scaffold; baseline (speedup 1.0000x reference)
import jax
import jax.numpy as jnp
from jax.experimental import pallas as pl


def kernel(x, neighborhood_matrix, W1, W2, W3, W4, W5, Wl, bl):
    raise NotImplementedError("write your pallas kernel here")


# speedup vs baseline: 8.6367x; 8.6367x over previous
import functools
import math

import jax
import jax.numpy as jnp
from jax import lax
from jax.experimental import pallas as pl
from jax.experimental.pallas import tpu as pltpu

B, N, K, D = 8, 1024, 32, 3
CH = 30
CP = 32
NEG = -1e30


def _prep_w(W, c_in):
    w = W.reshape(CH, K, c_in).transpose(1, 2, 0)
    w = jnp.pad(w, ((0, 0), (0, CP - c_in), (0, 0)), constant_values=NEG)
    w = jnp.pad(w, ((0, 0), (0, 0), (0, CP - CH)), constant_values=0.0)
    return w.astype(jnp.float32)


def _softmax_kc(w):
    m = jnp.max(jnp.max(w, axis=0, keepdims=True), axis=1, keepdims=True)
    e = jnp.exp(w - m)
    s = jnp.sum(jnp.sum(e, axis=0, keepdims=True), axis=1, keepdims=True)
    return e / s


ASIN_C = [math.factorial(2 * n) / (4 ** n * math.factorial(n) ** 2 * (2 * n + 1))
          for n in range(14)]


def _arccos(x):
    xa = jnp.abs(x)
    z2 = (1.0 - xa) * 0.5
    z = jnp.sqrt(z2)
    p = jnp.full_like(z2, ASIN_C[-1])
    for c in reversed(ASIN_C[:-1]):
        p = p * z2 + c
    ac_pos = 2.0 * (z * p)
    return jnp.where(x >= 0, ac_pos, math.pi - ac_pos)


def _nl96(acc):
    sq = acc * acc
    n2 = sq[:, 0:CP] + sq[:, CP:2 * CP] + sq[:, 2 * CP:3 * CP]
    n1 = jnp.sqrt(n2)
    inv1 = 1.0 / (n1 + 1e-8)
    nu = n1 * inv1
    scale = inv1 * (jnp.tanh(nu) / (nu + 1e-8))
    s3 = jnp.concatenate([scale, scale, scale], axis=1)
    return acc * s3


def _fused_kernel(h0_ref, nbrT_ref, w1_ref, w2_ref, w3_ref, w4_ref, w5_ref,
                  wl_ref, bl_ref, out_ref, wsm_ref):
    h = h0_ref[0]
    iota_j = lax.broadcasted_iota(jnp.int32, (N, N), 0)

    def wfm(h, w_ref):
        wsm_ref[...] = _softmax_kc(w_ref[...])
        h_bf = h.astype(jnp.bfloat16)

        def body(k, acc):
            nbr_k = nbrT_ref[0, pl.ds(k, 1), :]
            pkT = (iota_j == nbr_k).astype(jnp.bfloat16)
            dn = (((0,), (0,)), ((), ()))
            gk = lax.dot_general(pkT, h_bf, dn,
                                 preferred_element_type=jnp.float32)
            wk_bf = wsm_ref[k].astype(jnp.bfloat16)
            upd = [jnp.dot(gk[:, d * CP:(d + 1) * CP].astype(jnp.bfloat16),
                           wk_bf, preferred_element_type=jnp.float32)
                   for d in range(D)]
            return acc + jnp.concatenate(upd, axis=1)

        return lax.fori_loop(0, K, body, jnp.zeros((N, D * CP), jnp.float32))

    h = _nl96(wfm(h, w1_ref))
    h = _nl96(wfm(h, w2_ref))
    h = _nl96(wfm(h, w3_ref))
    h = _nl96(wfm(h, w4_ref))
    h = _nl96(wfm(h, w5_ref))

    hm = jnp.mean(h, axis=0, keepdims=True)
    msq = hm * hm
    mn = jnp.sqrt(msq[:, 0:CP] + msq[:, CP:2 * CP] + msq[:, 2 * CP:3 * CP])
    minv = 1.0 / (mn + 1e-8)
    hsq = h * h
    hn = jnp.sqrt(hsq[:, 0:CP] + hsq[:, CP:2 * CP] + hsq[:, 2 * CP:3 * CP])
    hinv = 1.0 / (hn + 1e-8)

    def bf(v):
        return v.astype(jnp.bfloat16).astype(jnp.float32)

    dots = (bf(h[:, 0:CP] * hinv) * bf(hm[:, 0:CP] * minv)
            + bf(h[:, CP:2 * CP] * hinv) * bf(hm[:, CP:2 * CP] * minv)
            + bf(h[:, 2 * CP:3 * CP] * hinv) * bf(hm[:, 2 * CP:3 * CP] * minv))
    dist = _arccos(jnp.clip(dots, -0.999, 0.999))
    feat = jnp.mean(dist, axis=0, keepdims=True)
    out_ref[0] = jnp.dot(feat.astype(jnp.bfloat16),
                         wl_ref[...].astype(jnp.bfloat16),
                         preferred_element_type=jnp.float32) + bl_ref[...]


@functools.partial(jax.jit, static_argnames=())
def kernel(x, neighborhood_matrix, W1, W2, W3, W4, W5, Wl, bl):
    x3 = x.reshape(B, N, D)
    h0 = jnp.zeros((B, N, D, CP), jnp.float32).at[:, :, :, 0].set(x3)
    h0 = h0.reshape(B, N, D * CP)
    nbrT = jnp.swapaxes(neighborhood_matrix, 1, 2).astype(jnp.int32)
    ws = [_prep_w(W1, 1)] + [_prep_w(W, CH) for W in (W2, W3, W4, W5)]
    wlp = jnp.pad(Wl, ((0, 0), (0, CP - CH))).T.astype(jnp.float32)
    blp = bl.reshape(1, -1).astype(jnp.float32)

    nc = Wl.shape[0]
    out = pl.pallas_call(
        _fused_kernel,
        grid=(B,),
        in_specs=[
            pl.BlockSpec((1, N, D * CP), lambda b: (b, 0, 0)),
            pl.BlockSpec((1, K, N), lambda b: (b, 0, 0)),
            pl.BlockSpec((K, CP, CP), lambda b: (0, 0, 0)),
            pl.BlockSpec((K, CP, CP), lambda b: (0, 0, 0)),
            pl.BlockSpec((K, CP, CP), lambda b: (0, 0, 0)),
            pl.BlockSpec((K, CP, CP), lambda b: (0, 0, 0)),
            pl.BlockSpec((K, CP, CP), lambda b: (0, 0, 0)),
            pl.BlockSpec((CP, nc), lambda b: (0, 0)),
            pl.BlockSpec((1, nc), lambda b: (0, 0)),
        ],
        out_specs=pl.BlockSpec((1, 1, nc), lambda b: (b, 0, 0)),
        out_shape=jax.ShapeDtypeStruct((B, 1, nc), jnp.float32),
        scratch_shapes=[pltpu.VMEM((K, CP, CP), jnp.float32)],
    )(h0, nbrT, ws[0], ws[1], ws[2], ws[3], ws[4], wlp, blp)
    return out.reshape(B, nc)


# speedup vs baseline: 10.2033x; 1.1814x over previous
import functools
import math

import jax
import jax.numpy as jnp
from jax import lax
from jax.experimental import pallas as pl
from jax.experimental.pallas import tpu as pltpu
from jax.experimental.pallas import tpu_sc as plsc

B, N, K, D = 8, 1024, 32, 3
CH = 30
CP = 32
RW = 4 * CP
NEG = -1e30
NC, NS = 2, 16
NKP = N * K


def _prep_w(W, c_in):
    w = W.reshape(CH, K, c_in).transpose(2, 1, 0)
    w = jnp.pad(w, ((0, CP - c_in), (0, 0), (0, 0)), constant_values=NEG)
    w = jnp.pad(w, ((0, 0), (0, 0), (0, CP - CH)), constant_values=0.0)
    return w.astype(jnp.float32)


def _softmax_ck(w):
    m = jnp.max(jnp.max(w, axis=0, keepdims=True), axis=1, keepdims=True)
    e = jnp.exp(w - m)
    s = jnp.sum(jnp.sum(e, axis=0, keepdims=True), axis=1, keepdims=True)
    return e / s


ASIN_C = [math.factorial(2 * n) / (4 ** n * math.factorial(n) ** 2 * (2 * n + 1))
          for n in range(14)]


def _arccos(x):
    xa = jnp.abs(x)
    z2 = (1.0 - xa) * 0.5
    z = jnp.sqrt(z2)
    p = jnp.full_like(z2, ASIN_C[-1])
    for c in reversed(ASIN_C[:-1]):
        p = p * z2 + c
    ac_pos = 2.0 * (z * p)
    return jnp.where(x >= 0, ac_pos, math.pi - ac_pos)


def _nl96(acc):
    sq = acc * acc
    n2 = sq[:, 0:CP] + sq[:, CP:2 * CP] + sq[:, 2 * CP:3 * CP]
    n1 = jnp.sqrt(n2)
    inv1 = 1.0 / (n1 + 1e-8)
    nu = n1 * inv1
    scale = inv1 * (jnp.tanh(nu) / (nu + 1e-8))
    s3 = jnp.concatenate([scale, scale, scale], axis=1)
    return acc * s3


def _layer_kernel(apply_nl, acc_ref, w_ref, t_ref):
    wsm = _softmax_ck(w_ref[...])
    z = jnp.zeros((CP, K, CP), jnp.float32)
    rows = [jnp.concatenate([wsm if dd == d else z for dd in range(D)] + [z],
                            axis=2)
            for d in range(D)]
    w3 = jnp.concatenate(rows, axis=0).reshape(D * CP, K * RW)
    w3_bf = w3.astype(jnp.bfloat16)
    h = acc_ref[0]
    if apply_nl:
        h = _nl96(h)
    t_ref[0] = jnp.dot(h.astype(jnp.bfloat16), w3_bf,
                       preferred_element_type=jnp.float32)


def _head_kernel(acc_ref, wl_ref, bl_ref, out_ref):
    h = _nl96(acc_ref[0])
    hm = jnp.mean(h, axis=0, keepdims=True)
    msq = hm * hm
    mn = jnp.sqrt(msq[:, 0:CP] + msq[:, CP:2 * CP] + msq[:, 2 * CP:3 * CP])
    minv = 1.0 / (mn + 1e-8)
    hsq = h * h
    hn = jnp.sqrt(hsq[:, 0:CP] + hsq[:, CP:2 * CP] + hsq[:, 2 * CP:3 * CP])
    hinv = 1.0 / (hn + 1e-8)

    def bf(v):
        return v.astype(jnp.bfloat16).astype(jnp.float32)

    dots = (bf(h[:, 0:CP] * hinv) * bf(hm[:, 0:CP] * minv)
            + bf(h[:, CP:2 * CP] * hinv) * bf(hm[:, CP:2 * CP] * minv)
            + bf(h[:, 2 * CP:3 * CP] * hinv) * bf(hm[:, 2 * CP:3 * CP] * minv))
    dist = _arccos(jnp.clip(dots, -0.999, 0.999))
    feat = jnp.mean(dist, axis=0, keepdims=True)
    out_ref[0] = jnp.dot(feat.astype(jnp.bfloat16),
                         wl_ref[...].astype(jnp.bfloat16),
                         preferred_element_type=jnp.float32) + bl_ref[...]


def _gather_kernel(t_ref, nbr_ref, out_ref, nbr_v, idx_v, rows_v, acc_v, sem):
    wid = lax.axis_index("s") * NC + lax.axis_index("c")
    n0 = wid * 32
    iota16 = lax.broadcasted_iota(jnp.int32, (16,), 0)

    def body_b(b, cb):
        pltpu.sync_copy(nbr_ref.at[b, pl.ds(n0, 32), :], nbr_v)
        base = b * NKP
        for c8 in range(8):
            for s in range(8):
                nrow = 4 * c8 + s // 2
                k0 = 16 * (s % 2)
                chunk = nbr_v[nrow, pl.ds(k0, 16)] * K + (iota16 + k0 + base)
                idx_v[c8, pl.ds(16 * s, 16)] = chunk
        for half in range(2):
            handles = [pltpu.async_copy(
                t_ref.at[idx_v.at[4 * half + c4]],
                rows_v.at[pl.ds(c4 * 128, 128), :], sem)
                for c4 in range(4)]
            for hnd in handles:
                hnd.wait()

            def body_p(p, cp):
                accs = [jnp.zeros((16,), jnp.float32) for _ in range(6)]
                for r in range(K):
                    for c in range(6):
                        accs[c] = accs[c] + rows_v[p * K + r, pl.ds(16 * c, 16)]
                for c in range(6):
                    acc_v[16 * half + p, pl.ds(16 * c, 16)] = accs[c]
                return cp

            lax.fori_loop(0, 16, body_p, 0)
        pltpu.sync_copy(acc_v, out_ref.at[b, pl.ds(n0, 32), :])
        return cb

    lax.fori_loop(0, B, body_b, 0)


def _tc_layer_call(apply_nl):
    return pl.pallas_call(
        functools.partial(_layer_kernel, apply_nl),
        grid=(B,),
        in_specs=[
            pl.BlockSpec((1, N, D * CP), lambda b: (b, 0, 0)),
            pl.BlockSpec((CP, K, CP), lambda b: (0, 0, 0)),
        ],
        out_specs=pl.BlockSpec((1, N, K * RW), lambda b: (b, 0, 0)),
        out_shape=jax.ShapeDtypeStruct((B, N, K * RW), jnp.float32),
    )


def _sc_gather_call():
    mesh = plsc.VectorSubcoreMesh(core_axis_name="c", subcore_axis_name="s")
    return functools.partial(
        pl.kernel,
        mesh=mesh,
        out_type=jax.ShapeDtypeStruct((B, N, D * CP), jnp.float32),
        scratch_types=[
            pltpu.VMEM((32, K), jnp.int32),
            pltpu.VMEM((8, 128), jnp.int32),
            pltpu.VMEM((512, RW), jnp.float32),
            pltpu.VMEM((32, D * CP), jnp.float32),
            pltpu.SemaphoreType.DMA,
        ],
    )(_gather_kernel)


def kernel(x, neighborhood_matrix, W1, W2, W3, W4, W5, Wl, bl):
    x3 = x.reshape(B, N, D)
    h0 = (jnp.zeros((B, N, D, CP), jnp.float32)
          .at[:, :, :, 0].set(x3).reshape(B, N, D * CP))
    nbr = neighborhood_matrix.astype(jnp.int32)
    ws = [_prep_w(W1, 1)] + [_prep_w(W, CH) for W in (W2, W3, W4, W5)]
    wlp = jnp.pad(Wl, ((0, 0), (0, CP - CH))).T.astype(jnp.float32)
    blp = bl.reshape(1, -1).astype(jnp.float32)
    nc = Wl.shape[0]

    layer1 = _tc_layer_call(False)
    layer = _tc_layer_call(True)
    gather = _sc_gather_call()

    t = layer1(h0, ws[0])
    acc = gather(t.reshape(B * N * K, RW), nbr)
    for w in ws[1:]:
        t = layer(acc, w)
        acc = gather(t.reshape(B * N * K, RW), nbr)

    out = pl.pallas_call(
        _head_kernel,
        grid=(B,),
        in_specs=[
            pl.BlockSpec((1, N, D * CP), lambda b: (b, 0, 0)),
            pl.BlockSpec((CP, nc), lambda b: (0, 0)),
            pl.BlockSpec((1, nc), lambda b: (0, 0)),
        ],
        out_specs=pl.BlockSpec((1, 1, nc), lambda b: (b, 0, 0)),
        out_shape=jax.ShapeDtypeStruct((B, 1, nc), jnp.float32),
    )(acc, wlp, blp)
    return out.reshape(B, nc)
